# baseline (speedup 1.0000x reference)
import functools

import jax
import jax.numpy as jnp
from jax.experimental import pallas as pl
from jax.experimental.pallas import tpu as pltpu

DIM = 1024
E = 8
FF = DIM * 4
TILE = 256
NB = 24
LANES = 128


def _router_body(x_ref, wr_ref, br_ref, routed_ref, lb_ref):
    x = x_ref[...]
    logits = jnp.dot(x, wr_ref[...], preferred_element_type=jnp.float32)
    T = x.shape[0]
    col = jax.lax.broadcasted_iota(jnp.int32, (T, LANES), 1)
    valid = col < E
    logits = jnp.where(valid, logits + br_ref[...], -1e30)
    m = jnp.max(logits, axis=1, keepdims=True)
    ex = jnp.exp(logits - m)
    p = ex / jnp.sum(ex, axis=1, keepdims=True)
    m1 = jnp.max(p, axis=1, keepdims=True)
    big = jnp.int32(LANES)
    i1 = jnp.min(jnp.where(p == m1, col, big), axis=1, keepdims=True)
    p2 = jnp.where(col == i1, -1.0, p)
    m2 = jnp.max(p2, axis=1, keepdims=True)
    i2 = jnp.min(jnp.where(p2 == m2, col, big), axis=1, keepdims=True)
    e2 = jnp.exp(m2 - m1)
    g1 = 1.0 / (1.0 + e2)
    g2 = e2 / (1.0 + e2)
    out = jnp.where(col == 0, i1.astype(jnp.float32), 0.0)
    out = jnp.where(col == 1, i2.astype(jnp.float32), out)
    out = jnp.where(col == 2, g1, out)
    out = jnp.where(col == 3, g2, out)
    routed_ref[...] = out
    cnt = (jnp.sum((i1 == col).astype(jnp.float32), axis=0)
           + jnp.sum((i2 == col).astype(jnp.float32), axis=0))
    load = cnt / (2.0 * T)
    contrib = jnp.where(col[0] < E, (load - 1.0 / E) ** 2, 0.0)
    lb_ref[0, 0] = jnp.sum(contrib)


def _router(xf, wr_t, br_pad):
    T = xf.shape[0]
    routed, lb = pl.pallas_call(
        _router_body,
        out_shape=(
            jax.ShapeDtypeStruct((T, LANES), jnp.float32),
            jax.ShapeDtypeStruct((1, 1), jnp.float32),
        ),
        in_specs=[
            pl.BlockSpec((T, DIM), lambda: (0, 0)),
            pl.BlockSpec((DIM, LANES), lambda: (0, 0)),
            pl.BlockSpec((1, LANES), lambda: (0, 0)),
        ],
        out_specs=(
            pl.BlockSpec((T, LANES), lambda: (0, 0)),
            pl.BlockSpec(memory_space=pltpu.SMEM),
        ),
    )(xf, wr_t, br_pad)
    return routed, lb


KF = 2
FFT = FF // KF
MAX_TPE = 2048 // TILE


def _moe_body(ntiles_ref, toff_ref, tok_ref,
              x_ref, w1_ref, b1_ref, w2_ref, b2_ref, gate_ref,
              out_ref, xg_ref, y_ref, acc_ref):
    k = pl.program_id(0)
    e = pl.program_id(1)

    @pl.when(jnp.logical_and(k == 0, e == 0))
    def _zero():
        out_ref[...] = jnp.zeros_like(out_ref)

    def tile_body(j, _):
        t = toff_ref[e] + j

        def gather(i, _):
            xg_ref[i, :] = x_ref[tok_ref[t, i], :]
            return 0
        jax.lax.fori_loop(0, TILE, gather, 0, unroll=8)

        h = jnp.dot(xg_ref[...], w1_ref[0], preferred_element_type=jnp.float32)
        h = h + b1_ref[0]
        h = 0.5 * h * (1.0 + jax.lax.erf(h * 0.7071067811865476))
        y = jnp.dot(h, w2_ref[0], preferred_element_type=jnp.float32)
        y = jnp.where(k == 0, y + b2_ref[0], y)

        def rd(i, _):
            acc_ref[i, :] = out_ref[tok_ref[t, i], :]
            return 0
        jax.lax.fori_loop(0, TILE, rd, 0, unroll=8)

        y_ref[...] = acc_ref[...] + y * gate_ref[t, 0][:, None]

        def wr(i, _):
            out_ref[tok_ref[t, i], :] = y_ref[i, :]
            return 0
        jax.lax.fori_loop(0, TILE, wr, 0, unroll=8)
        return 0

    jax.lax.fori_loop(0, ntiles_ref[e], tile_body, 0)


def _moe(xf, w1, b1, w2, b2, ntiles, toff, slot_tok, slot_gate):
    T = xf.shape[0]
    grid_spec = pltpu.PrefetchScalarGridSpec(
        num_scalar_prefetch=3,
        grid=(KF, E),
        in_specs=[
            pl.BlockSpec((T, DIM), lambda k, e, nt, to, tok: (0, 0)),
            pl.BlockSpec((1, DIM, FFT), lambda k, e, nt, to, tok: (e, 0, k)),
            pl.BlockSpec((1, 1, FFT), lambda k, e, nt, to, tok: (e, 0, k)),
            pl.BlockSpec((1, FFT, DIM), lambda k, e, nt, to, tok: (e, k, 0)),
            pl.BlockSpec((1, 1, DIM), lambda k, e, nt, to, tok: (e, 0, 0)),
            pl.BlockSpec((NB, 1, TILE), lambda k, e, nt, to, tok: (0, 0, 0)),
        ],
        out_specs=pl.BlockSpec((T, DIM), lambda k, e, nt, to, tok: (0, 0)),
        scratch_shapes=[
            pltpu.VMEM((TILE, DIM), jnp.float32),
            pltpu.VMEM((TILE, DIM), jnp.float32),
            pltpu.VMEM((TILE, DIM), jnp.float32),
        ],
    )
    return pl.pallas_call(
        _moe_body,
        grid_spec=grid_spec,
        out_shape=jax.ShapeDtypeStruct((T, DIM), jnp.float32),
    )(ntiles, toff, slot_tok, xf, w1,
      b1.reshape(E, 1, FF), w2, b2.reshape(E, 1, DIM),
      slot_gate.reshape(NB, 1, TILE))


@functools.partial(jax.jit, static_argnames=())
def kernel(x, Wr, br, W1, b1, W2, b2):
    b, s, d = x.shape
    T = b * s
    xf = x.reshape(T, d)

    wr_t = jnp.zeros((DIM, LANES), jnp.float32).at[:, :E].set(Wr.T)
    br_pad = jnp.zeros((1, LANES), jnp.float32).at[0, :E].set(br)
    routed, lb = _router(xf, wr_t, br_pad)

    idx1 = routed[:, 0].astype(jnp.int32)
    idx2 = routed[:, 1].astype(jnp.int32)
    g1 = routed[:, 2]
    g2 = routed[:, 3]

    expert_flat = jnp.concatenate([idx1, idx2])
    tok_flat = jnp.concatenate([jnp.arange(T, dtype=jnp.int32)] * 2)
    gate_flat = jnp.concatenate([g1, g2])
    oh = (expert_flat[:, None] == jnp.arange(E)[None, :]).astype(jnp.int32)
    pos = jnp.cumsum(oh, axis=0) - 1
    pos_own = jnp.take_along_axis(pos, expert_flat[:, None], axis=1)[:, 0]
    counts = oh.sum(axis=0)
    padded = ((counts + TILE - 1) // TILE) * TILE
    cum = jnp.cumsum(padded)
    offs = cum - padded
    slot = offs[expert_flat] + pos_own
    slot_tok = jnp.zeros((NB * TILE,), jnp.int32).at[slot].set(tok_flat)
    slot_gate = jnp.zeros((NB * TILE,), jnp.float32).at[slot].set(gate_flat)
    ntiles = (padded // TILE).astype(jnp.int32)
    toff = (offs // TILE).astype(jnp.int32)

    out = _moe(xf, W1, b1, W2, b2,
               ntiles, toff,
               slot_tok.reshape(NB, TILE),
               slot_gate.reshape(NB, TILE))
    return out.reshape(b, s, d), lb[0, 0]


# speedup vs baseline: 1.2541x; 1.2541x over previous
import functools

import jax
import jax.numpy as jnp
from jax.experimental import pallas as pl
from jax.experimental.pallas import tpu as pltpu

DIM = 1024
E = 8
FF = DIM * 4
TILE = 256
NB = 24
LANES = 128


def _cumsum_rows(m, T):
    c = m
    sh = 1
    while sh < T:
        c = c + jnp.pad(c, ((sh, 0), (0, 0)))[:T]
        sh *= 2
    return c


def _router_body(x_ref, wr_ref, br_ref, routed_ref, sched_ref, lb_ref):
    x = x_ref[...]
    logits = jnp.dot(x, wr_ref[...], preferred_element_type=jnp.float32)
    T = x.shape[0]
    col = jax.lax.broadcasted_iota(jnp.int32, (T, LANES), 1)
    valid = col < E
    logits = jnp.where(valid, logits + br_ref[...], -1e30)
    m = jnp.max(logits, axis=1, keepdims=True)
    ex = jnp.exp(logits - m)
    p = ex / jnp.sum(ex, axis=1, keepdims=True)
    m1 = jnp.max(p, axis=1, keepdims=True)
    big = jnp.int32(LANES)
    i1 = jnp.min(jnp.where(p == m1, col, big), axis=1, keepdims=True)
    p2 = jnp.where(col == i1, -1.0, p)
    m2 = jnp.max(p2, axis=1, keepdims=True)
    i2 = jnp.min(jnp.where(p2 == m2, col, big), axis=1, keepdims=True)
    e2 = jnp.exp(m2 - m1)
    g1 = 1.0 / (1.0 + e2)
    g2 = e2 / (1.0 + e2)

    mask1 = (col == i1).astype(jnp.float32)
    mask2 = (col == i2).astype(jnp.float32)
    c1 = _cumsum_rows(mask1, T)
    c2 = _cumsum_rows(mask2, T)
    counts1 = c1[T - 1:T, :]
    counts2 = c2[T - 1:T, :]
    counts = counts1 + counts2
    inv_tile = 1.0 / TILE
    padded = jnp.ceil(counts * inv_tile) * TILE
    ntiles = padded * inv_tile
    ltm = (jax.lax.broadcasted_iota(jnp.int32, (LANES, LANES), 0)
           < jax.lax.broadcasted_iota(jnp.int32, (LANES, LANES), 1)
           ).astype(jnp.float32)
    offs = jnp.dot(padded, ltm, preferred_element_type=jnp.float32)
    pos1 = jnp.sum(mask1 * (c1 - 1.0), axis=1, keepdims=True)
    slot1 = jnp.sum(mask1 * offs, axis=1, keepdims=True) + pos1
    pos2 = jnp.sum(mask2 * (c2 - 1.0 + counts1), axis=1, keepdims=True)
    slot2 = jnp.sum(mask2 * offs, axis=1, keepdims=True) + pos2

    out = jnp.where(col == 0, slot1, 0.0)
    out = jnp.where(col == 1, slot2, out)
    out = jnp.where(col == 2, g1, out)
    out = jnp.where(col == 3, g2, out)
    routed_ref[...] = out
    sched = jnp.concatenate([ntiles, offs * inv_tile], axis=0)
    sched_ref[...] = sched
    load = counts[0] / (2.0 * T)
    contrib = jnp.where(col[0] < E, (load - 1.0 / E) ** 2, 0.0)
    lb_ref[0, 0] = jnp.sum(contrib)


def _router(xf, wr_t, br_pad):
    T = xf.shape[0]
    routed, sched, lb = pl.pallas_call(
        _router_body,
        out_shape=(
            jax.ShapeDtypeStruct((T, LANES), jnp.float32),
            jax.ShapeDtypeStruct((2, LANES), jnp.float32),
            jax.ShapeDtypeStruct((1, 1), jnp.float32),
        ),
        in_specs=[
            pl.BlockSpec((T, DIM), lambda: (0, 0)),
            pl.BlockSpec((DIM, LANES), lambda: (0, 0)),
            pl.BlockSpec((1, LANES), lambda: (0, 0)),
        ],
        out_specs=(
            pl.BlockSpec((T, LANES), lambda: (0, 0)),
            pl.BlockSpec((2, LANES), lambda: (0, 0)),
            pl.BlockSpec(memory_space=pltpu.SMEM),
        ),
    )(xf, wr_t, br_pad)
    return routed, sched, lb


KF = 2
FFT = FF // KF
MAX_TPE = 2048 // TILE


def _moe_body(ntiles_ref, toff_ref, tok_ref,
              x_ref, w1_ref, b1_ref, w2_ref, b2_ref, gate_ref,
              out_ref, xg_ref, y_ref):
    k = pl.program_id(0)
    e = pl.program_id(1)

    @pl.when(jnp.logical_and(k == 0, e == 0))
    def _zero():
        out_ref[...] = jnp.zeros_like(out_ref)

    def tile_body(j, _):
        t = toff_ref[e] + j

        def gather(i, _):
            xg_ref[i, :] = x_ref[tok_ref[t, i], :]
            return 0
        jax.lax.fori_loop(0, TILE, gather, 0, unroll=8)

        h = jnp.dot(xg_ref[...], w1_ref[0], preferred_element_type=jnp.float32)
        h = h + b1_ref[0]
        h = 0.5 * h * (1.0 + jax.lax.erf(h * 0.7071067811865476))
        y = jnp.dot(h, w2_ref[0], preferred_element_type=jnp.float32)
        y = jnp.where(k == 0, y + b2_ref[0], y)
        y_ref[...] = y * gate_ref[t, 0][:, None]

        def scatter(i, _):
            tok = tok_ref[t, i]
            out_ref[tok, :] = out_ref[tok, :] + y_ref[i, :]
            return 0
        jax.lax.fori_loop(0, TILE, scatter, 0, unroll=8)
        return 0

    jax.lax.fori_loop(0, ntiles_ref[e], tile_body, 0)


def _moe(xf, w1, b1, w2, b2, ntiles, toff, slot_tok, slot_gate):
    T = xf.shape[0]
    grid_spec = pltpu.PrefetchScalarGridSpec(
        num_scalar_prefetch=3,
        grid=(KF, E),
        in_specs=[
            pl.BlockSpec((T, DIM), lambda k, e, nt, to, tok: (0, 0)),
            pl.BlockSpec((1, DIM, FFT), lambda k, e, nt, to, tok: (e, 0, k)),
            pl.BlockSpec((1, 1, FFT), lambda k, e, nt, to, tok: (e, 0, k)),
            pl.BlockSpec((1, FFT, DIM), lambda k, e, nt, to, tok: (e, k, 0)),
            pl.BlockSpec((1, 1, DIM), lambda k, e, nt, to, tok: (e, 0, 0)),
            pl.BlockSpec((NB, 1, TILE), lambda k, e, nt, to, tok: (0, 0, 0)),
        ],
        out_specs=pl.BlockSpec((T, DIM), lambda k, e, nt, to, tok: (0, 0)),
        scratch_shapes=[
            pltpu.VMEM((TILE, DIM), jnp.float32),
            pltpu.VMEM((TILE, DIM), jnp.float32),
        ],
    )
    return pl.pallas_call(
        _moe_body,
        grid_spec=grid_spec,
        out_shape=jax.ShapeDtypeStruct((T, DIM), jnp.float32),
    )(ntiles, toff, slot_tok, xf, w1,
      b1.reshape(E, 1, FF), w2, b2.reshape(E, 1, DIM),
      slot_gate.reshape(NB, 1, TILE))


@functools.partial(jax.jit, static_argnames=())
def kernel(x, Wr, br, W1, b1, W2, b2):
    b, s, d = x.shape
    T = b * s
    xf = x.reshape(T, d)

    wr_t = jnp.zeros((DIM, LANES), jnp.float32).at[:, :E].set(Wr.T)
    br_pad = jnp.zeros((1, LANES), jnp.float32).at[0, :E].set(br)
    routed, sched, lb = _router(xf, wr_t, br_pad)

    slot1 = routed[:, 0].astype(jnp.int32)
    slot2 = routed[:, 1].astype(jnp.int32)
    ar = jnp.arange(T, dtype=jnp.int32)
    slot_tok = (jnp.zeros((NB * TILE,), jnp.int32)
                .at[slot1].set(ar, unique_indices=True)
                .at[slot2].set(ar, unique_indices=True))
    slot_gate = (jnp.zeros((NB * TILE,), jnp.float32)
                 .at[slot1].set(routed[:, 2], unique_indices=True)
                 .at[slot2].set(routed[:, 3], unique_indices=True))
    ntiles = sched[0, :E].astype(jnp.int32)
    toff = sched[1, :E].astype(jnp.int32)

    out = _moe(xf, W1, b1, W2, b2,
               ntiles, toff,
               slot_tok.reshape(NB, TILE),
               slot_gate.reshape(NB, TILE))
    return out.reshape(b, s, d), lb[0, 0]


# speedup vs baseline: 1.2728x; 1.0149x over previous
import functools

import jax
import jax.numpy as jnp
from jax.experimental import pallas as pl
from jax.experimental.pallas import tpu as pltpu

DIM = 1024
E = 8
FF = DIM * 4
TILE = 256
NB = 24
LANES = 128


def _cumsum_rows(m, T):
    c = m
    sh = 1
    while sh < T:
        c = c + jnp.pad(c, ((sh, 0), (0, 0)))[:T]
        sh *= 2
    return c


def _router_body(x_ref, wr_ref, br_ref, routed_ref, sched_ref, lb_ref):
    x = x_ref[...]
    logits = jnp.dot(x, wr_ref[...], preferred_element_type=jnp.float32)
    T = x.shape[0]
    col = jax.lax.broadcasted_iota(jnp.int32, (T, LANES), 1)
    valid = col < E
    logits = jnp.where(valid, logits + br_ref[...], -1e30)
    m = jnp.max(logits, axis=1, keepdims=True)
    ex = jnp.exp(logits - m)
    p = ex / jnp.sum(ex, axis=1, keepdims=True)
    m1 = jnp.max(p, axis=1, keepdims=True)
    big = jnp.int32(LANES)
    i1 = jnp.min(jnp.where(p == m1, col, big), axis=1, keepdims=True)
    p2 = jnp.where(col == i1, -1.0, p)
    m2 = jnp.max(p2, axis=1, keepdims=True)
    i2 = jnp.min(jnp.where(p2 == m2, col, big), axis=1, keepdims=True)
    e2 = jnp.exp(m2 - m1)
    g1 = 1.0 / (1.0 + e2)
    g2 = e2 / (1.0 + e2)

    mask1 = (col == i1).astype(jnp.float32)
    mask2 = (col == i2).astype(jnp.float32)
    c1 = _cumsum_rows(mask1, T)
    c2 = _cumsum_rows(mask2, T)
    counts1 = c1[T - 1:T, :]
    counts2 = c2[T - 1:T, :]
    counts = counts1 + counts2
    inv_tile = 1.0 / TILE
    padded = jnp.ceil(counts * inv_tile) * TILE
    ntiles = padded * inv_tile
    ltm = (jax.lax.broadcasted_iota(jnp.int32, (LANES, LANES), 0)
           < jax.lax.broadcasted_iota(jnp.int32, (LANES, LANES), 1)
           ).astype(jnp.float32)
    offs = jnp.dot(padded, ltm, preferred_element_type=jnp.float32)
    pos1 = jnp.sum(mask1 * (c1 - 1.0), axis=1, keepdims=True)
    slot1 = jnp.sum(mask1 * offs, axis=1, keepdims=True) + pos1
    pos2 = jnp.sum(mask2 * (c2 - 1.0 + counts1), axis=1, keepdims=True)
    slot2 = jnp.sum(mask2 * offs, axis=1, keepdims=True) + pos2

    out = jnp.where(col == 0, slot1, 0.0)
    out = jnp.where(col == 1, slot2, out)
    out = jnp.where(col == 2, g1, out)
    out = jnp.where(col == 3, g2, out)
    routed_ref[...] = out
    sched = jnp.concatenate([ntiles, offs * inv_tile], axis=0)
    sched_ref[...] = sched
    load = counts[0] / (2.0 * T)
    contrib = jnp.where(col[0] < E, (load - 1.0 / E) ** 2, 0.0)
    lb_ref[0, 0] = jnp.sum(contrib)


def _router(xf, wr_t, br_pad):
    T = xf.shape[0]
    routed, sched, lb = pl.pallas_call(
        _router_body,
        out_shape=(
            jax.ShapeDtypeStruct((T, LANES), jnp.float32),
            jax.ShapeDtypeStruct((2, LANES), jnp.float32),
            jax.ShapeDtypeStruct((1, 1), jnp.float32),
        ),
        in_specs=[
            pl.BlockSpec((T, DIM), lambda: (0, 0)),
            pl.BlockSpec((DIM, LANES), lambda: (0, 0)),
            pl.BlockSpec((1, LANES), lambda: (0, 0)),
        ],
        out_specs=(
            pl.BlockSpec((T, LANES), lambda: (0, 0)),
            pl.BlockSpec((2, LANES), lambda: (0, 0)),
            pl.BlockSpec(memory_space=pltpu.SMEM),
        ),
    )(xf, wr_t, br_pad)
    return routed, sched, lb


KF = 2
FFT = FF // KF
MAX_TPE = 2048 // TILE


def _moe_body(ntiles_ref, toff_ref, tok_ref,
              x_ref, w1_ref, b1_ref, w2_ref, b2_ref, gate_ref,
              out_ref, xg_ref, y_ref):
    k = pl.program_id(0)
    e = pl.program_id(1)

    @pl.when(jnp.logical_and(k == 0, e == 0))
    def _zero():
        out_ref[...] = jnp.zeros_like(out_ref)

    def tile_body(j, _):
        t = toff_ref[e] + j

        def gather(i, _):
            xg_ref[i, :] = x_ref[tok_ref[t, i], :]
            return 0
        jax.lax.fori_loop(0, TILE, gather, 0, unroll=8)

        h = jnp.dot(xg_ref[...], w1_ref[0], preferred_element_type=jnp.float32)
        h = h + b1_ref[0]
        h = 0.5 * h * (1.0 + jax.lax.erf(h * 0.7071067811865476))
        y = jnp.dot(h, w2_ref[0], preferred_element_type=jnp.float32)
        y = jnp.where(k == 0, y + b2_ref[0], y)
        y_ref[...] = y * gate_ref[t, 0][:, None]

        def scatter(i, _):
            tok = tok_ref[t, i]
            out_ref[tok, :] = out_ref[tok, :] + y_ref[i, :]
            return 0
        jax.lax.fori_loop(0, TILE, scatter, 0, unroll=8)
        return 0

    jax.lax.fori_loop(0, ntiles_ref[e], tile_body, 0)


def _moe(xf, w1, b1, w2, b2, ntiles, toff, slot_tok, slot_gate):
    T = xf.shape[0]
    grid_spec = pltpu.PrefetchScalarGridSpec(
        num_scalar_prefetch=3,
        grid=(KF, E),
        in_specs=[
            pl.BlockSpec((T, DIM), lambda k, e, nt, to, tok: (0, 0)),
            pl.BlockSpec((1, DIM, FFT), lambda k, e, nt, to, tok: (e, 0, k)),
            pl.BlockSpec((1, 1, FFT), lambda k, e, nt, to, tok: (e, 0, k)),
            pl.BlockSpec((1, FFT, DIM), lambda k, e, nt, to, tok: (e, k, 0)),
            pl.BlockSpec((1, 1, DIM), lambda k, e, nt, to, tok: (e, 0, 0)),
            pl.BlockSpec((NB, 1, TILE), lambda k, e, nt, to, tok: (0, 0, 0)),
        ],
        out_specs=pl.BlockSpec((T, DIM), lambda k, e, nt, to, tok: (0, 0)),
        scratch_shapes=[
            pltpu.VMEM((TILE, DIM), jnp.float32),
            pltpu.VMEM((TILE, DIM), jnp.float32),
        ],
    )
    return pl.pallas_call(
        _moe_body,
        grid_spec=grid_spec,
        out_shape=jax.ShapeDtypeStruct((T, DIM), jnp.float32),
    )(ntiles, toff, slot_tok, xf, w1,
      b1.reshape(E, 1, FF), w2, b2.reshape(E, 1, DIM),
      slot_gate.reshape(NB, 1, TILE))


@functools.partial(jax.jit, static_argnames=())
def kernel(x, Wr, br, W1, b1, W2, b2):
    b, s, d = x.shape
    T = b * s
    xf = x.reshape(T, d)

    wr_t = jnp.zeros((DIM, LANES), jnp.float32).at[:, :E].set(Wr.T)
    br_pad = jnp.zeros((1, LANES), jnp.float32).at[0, :E].set(br)
    routed, sched, lb = _router(xf, wr_t, br_pad)

    slots = routed[:, 0:2].astype(jnp.int32).T.reshape(-1)
    ar = jnp.arange(T, dtype=jnp.int32)
    slot_tok = (jnp.zeros((NB * TILE,), jnp.int32)
                .at[slots].set(jnp.concatenate([ar, ar]),
                               unique_indices=True,
                               mode="promise_in_bounds"))
    slot_gate = (jnp.zeros((NB * TILE,), jnp.float32)
                 .at[slots].set(routed[:, 2:4].T.reshape(-1),
                                unique_indices=True,
                                mode="promise_in_bounds"))
    ntiles = sched[0, :E].astype(jnp.int32)
    toff = sched[1, :E].astype(jnp.int32)

    out = _moe(xf, W1, b1, W2, b2,
               ntiles, toff,
               slot_tok.reshape(NB, TILE),
               slot_gate.reshape(NB, TILE))
    return out.reshape(b, s, d), lb[0, 0]
